# initial kernel scaffold (unmeasured)
import jax
import jax.numpy as jnp
from jax import lax
from jax.experimental import pallas as pl
from jax.experimental.pallas import tpu as pltpu

N_DEV = 4
BM = 1024
BK = 1024
N = 8192
BN = 1024
NN = N // BN

_DeviceIdType = getattr(pl, "DeviceIdType", None) or pltpu.DeviceIdType
_sem_signal = getattr(pl, "semaphore_signal", None) or pltpu.semaphore_signal
_sem_wait = getattr(pl, "semaphore_wait", None) or pltpu.semaphore_wait
_CompilerParams = getattr(pltpu, "CompilerParams", None) or pltpu.TPUCompilerParams


def _body(x_ref, w_ref, out_ref, xrecv, send_sems, recv_sems, copy_sem):
    s = pl.program_id(0)
    n = pl.program_id(1)
    my_i = lax.axis_index("i")

    @pl.when(jnp.logical_and(s == 0, n == 0))
    def _init():
        barrier = pltpu.get_barrier_semaphore()
        for d in (1, 2, 3):
            _sem_signal(
                barrier, inc=1,
                device_id=((my_i + d) % N_DEV,),
                device_id_type=_DeviceIdType.MESH,
            )
        _sem_wait(barrier, 3)

        cp = pltpu.make_async_copy(
            x_ref.at[pl.ds(my_i * BM, BM), :], xrecv.at[my_i], copy_sem
        )
        cp.start()

        for d in (1, 3, 2):
            tgt = (my_i + d) % N_DEV
            pltpu.make_async_remote_copy(
                src_ref=x_ref.at[pl.ds(tgt * BM, BM), :],
                dst_ref=xrecv.at[my_i],
                send_sem=send_sems.at[tgt],
                recv_sem=recv_sems.at[my_i],
                device_id=(tgt,),
                device_id_type=_DeviceIdType.MESH,
            ).start()
        cp.wait()

    @pl.when(jnp.logical_and(n == 0, s != my_i))
    def _wait_block():
        pltpu.make_async_remote_copy(
            src_ref=x_ref.at[pl.ds(0, BM), :],
            dst_ref=xrecv.at[s],
            send_sem=send_sems.at[0],
            recv_sem=recv_sems.at[s],
            device_id=(my_i,),
            device_id_type=_DeviceIdType.MESH,
        ).wait_recv()

    xb = xrecv[s]
    wb = w_ref[...].astype(jnp.bfloat16)
    acc = jnp.dot(xb, wb, preferred_element_type=jnp.float32)
    cols = pl.ds(n * BN, BN)

    @pl.when(s == 0)
    def _store():
        out_ref[:, cols] = acc

    @pl.when(s != 0)
    def _accum():
        out_ref[:, cols] = out_ref[:, cols] + acc

    @pl.when(jnp.logical_and(s == N_DEV - 1, n == NN - 1))
    def _fin():
        for d in (1, 2, 3):
            tgt = (my_i + d) % N_DEV
            pltpu.make_async_remote_copy(
                src_ref=x_ref.at[pl.ds(tgt * BM, BM), :],
                dst_ref=xrecv.at[my_i],
                send_sem=send_sems.at[tgt],
                recv_sem=recv_sems.at[my_i],
                device_id=(tgt,),
                device_id_type=_DeviceIdType.MESH,
            ).wait_send()


def kernel(x, w_mat):
    xb = x.astype(jnp.bfloat16)
    return pl.pallas_call(
        _body,
        grid=(N_DEV, NN),
        in_specs=[
            pl.BlockSpec(memory_space=pltpu.VMEM),
            pl.BlockSpec((BK, BN), lambda s, n: (s, n)),
        ],
        out_specs=pl.BlockSpec(memory_space=pltpu.VMEM),
        out_shape=jax.ShapeDtypeStruct((BM, N), jnp.float32),
        scratch_shapes=[
            pltpu.VMEM((N_DEV, BM, BK), jnp.bfloat16),
            pltpu.SemaphoreType.DMA((N_DEV,)),
            pltpu.SemaphoreType.DMA((N_DEV,)),
            pltpu.SemaphoreType.DMA,
        ],
        compiler_params=_CompilerParams(
            collective_id=0,
            dimension_semantics=("arbitrary", "arbitrary"),
        ),
    )(xb, w_mat)


# baseline (device time: 188695 ns/iter reference)
import jax
import jax.numpy as jnp
from jax import lax
from jax.experimental import pallas as pl
from jax.experimental.pallas import tpu as pltpu

N_DEV = 4
BM = 1024
BK = 1024
N = 8192
BN = 1024
NN = N // BN

_DeviceIdType = getattr(pl, "DeviceIdType", None) or pltpu.DeviceIdType
_sem_signal = getattr(pl, "semaphore_signal", None) or pltpu.semaphore_signal
_sem_wait = getattr(pl, "semaphore_wait", None) or pltpu.semaphore_wait
_CompilerParams = getattr(pltpu, "CompilerParams", None) or pltpu.TPUCompilerParams


def _body(x_ref, w_ref, out_ref, xrecv, send_sems, recv_sems, copy_sem):
    s = pl.program_id(0)
    n = pl.program_id(1)
    my_i = lax.axis_index("i")

    @pl.when(jnp.logical_and(s == 0, n == 0))
    def _init():
        barrier = pltpu.get_barrier_semaphore()
        for d in (1, 2, 3):
            _sem_signal(
                barrier, inc=1,
                device_id=((my_i + d) % N_DEV,),
                device_id_type=_DeviceIdType.MESH,
            )
        _sem_wait(barrier, 3)

        cp = pltpu.make_async_copy(
            x_ref.at[pl.ds(my_i * BM, BM), :], xrecv.at[my_i], copy_sem
        )
        cp.start()

        for d in (1, 3, 2):
            tgt = (my_i + d) % N_DEV
            pltpu.make_async_remote_copy(
                src_ref=x_ref.at[pl.ds(tgt * BM, BM), :],
                dst_ref=xrecv.at[my_i],
                send_sem=send_sems.at[tgt],
                recv_sem=recv_sems.at[my_i],
                device_id=(tgt,),
                device_id_type=_DeviceIdType.MESH,
            ).start()
        cp.wait()

    @pl.when(jnp.logical_and(n == 0, s != my_i))
    def _wait_block():
        pltpu.make_async_remote_copy(
            src_ref=x_ref.at[pl.ds(0, BM), :],
            dst_ref=xrecv.at[s],
            send_sem=send_sems.at[0],
            recv_sem=recv_sems.at[s],
            device_id=(my_i,),
            device_id_type=_DeviceIdType.MESH,
        ).wait_recv()

    xb = xrecv[s]
    wb = w_ref[...].astype(jnp.bfloat16)
    acc = jnp.dot(xb, wb, preferred_element_type=jnp.float32)
    cols = pl.ds(n * BN, BN)

    @pl.when(s == 0)
    def _store():
        out_ref[:, cols] = acc

    @pl.when(s != 0)
    def _accum():
        out_ref[:, cols] = out_ref[:, cols] + acc

    @pl.when(jnp.logical_and(s == N_DEV - 1, n == NN - 1))
    def _fin():
        for d in (1, 2, 3):
            tgt = (my_i + d) % N_DEV
            pltpu.make_async_remote_copy(
                src_ref=x_ref.at[pl.ds(tgt * BM, BM), :],
                dst_ref=xrecv.at[my_i],
                send_sem=send_sems.at[tgt],
                recv_sem=recv_sems.at[my_i],
                device_id=(tgt,),
                device_id_type=_DeviceIdType.MESH,
            ).wait_send()


def kernel(x, w_mat):
    xb = x.astype(jnp.bfloat16)
    return pl.pallas_call(
        _body,
        grid=(N_DEV, NN),
        in_specs=[
            pl.BlockSpec(memory_space=pltpu.VMEM),
            pl.BlockSpec((BK, BN), lambda s, n: (s, n)),
        ],
        out_specs=pl.BlockSpec(memory_space=pltpu.VMEM),
        out_shape=jax.ShapeDtypeStruct((BM, N), jnp.float32),
        scratch_shapes=[
            pltpu.VMEM((N_DEV, BM, BK), jnp.bfloat16),
            pltpu.SemaphoreType.DMA((N_DEV,)),
            pltpu.SemaphoreType.DMA((N_DEV,)),
            pltpu.SemaphoreType.DMA,
        ],
        compiler_params=_CompilerParams(
            collective_id=0,
            dimension_semantics=("arbitrary", "arbitrary"),
            vmem_limit_bytes=62 * 1024 * 1024,
        ),
    )(xb, w_mat)


# device time: 153748 ns/iter; 1.2273x vs baseline; 1.2273x over previous
import jax
import jax.numpy as jnp
from jax import lax
from jax.experimental import pallas as pl
from jax.experimental.pallas import tpu as pltpu

N_DEV = 4
BM = 1024
BK = 1024
N = 8192
BN = 1024
NN = N // BN

_DeviceIdType = getattr(pl, "DeviceIdType", None) or pltpu.DeviceIdType
_sem_signal = getattr(pl, "semaphore_signal", None) or pltpu.semaphore_signal
_sem_wait = getattr(pl, "semaphore_wait", None) or pltpu.semaphore_wait
_CompilerParams = getattr(pltpu, "CompilerParams", None) or pltpu.TPUCompilerParams


def _body(perm_ref, x_ref, w_ref, out_ref, xrecv, send_sems, recv_sems, copy_sem):
    s = pl.program_id(0)
    n = pl.program_id(1)
    my_i = lax.axis_index("i")
    j = perm_ref[s]

    @pl.when(jnp.logical_and(s == 0, n == 0))
    def _init():
        barrier = pltpu.get_barrier_semaphore()
        for d in (1, 2, 3):
            _sem_signal(
                barrier, inc=1,
                device_id=((my_i + d) % N_DEV,),
                device_id_type=_DeviceIdType.MESH,
            )
        _sem_wait(barrier, 3)

        cp = pltpu.make_async_copy(
            x_ref.at[pl.ds(my_i * BM, BM), :], xrecv.at[my_i], copy_sem
        )
        cp.start()

        for d in (3, 2, 1):
            tgt = (my_i + d) % N_DEV
            pltpu.make_async_remote_copy(
                src_ref=x_ref.at[pl.ds(tgt * BM, BM), :],
                dst_ref=xrecv.at[my_i],
                send_sem=send_sems.at[tgt],
                recv_sem=recv_sems.at[my_i],
                device_id=(tgt,),
                device_id_type=_DeviceIdType.MESH,
            ).start()
        cp.wait()

    @pl.when(jnp.logical_and(n == 0, s != 0))
    def _wait_block():
        pltpu.make_async_remote_copy(
            src_ref=x_ref.at[pl.ds(0, BM), :],
            dst_ref=xrecv.at[j],
            send_sem=send_sems.at[0],
            recv_sem=recv_sems.at[j],
            device_id=(my_i,),
            device_id_type=_DeviceIdType.MESH,
        ).wait_recv()

    xb = xrecv[j]
    wb = w_ref[...].astype(jnp.bfloat16)
    acc = jnp.dot(xb, wb, preferred_element_type=jnp.float32)
    cols = pl.ds(n * BN, BN)

    @pl.when(s == 0)
    def _store():
        out_ref[:, cols] = acc

    @pl.when(s != 0)
    def _accum():
        out_ref[:, cols] = out_ref[:, cols] + acc

    @pl.when(jnp.logical_and(s == N_DEV - 1, n == NN - 1))
    def _fin():
        for d in (1, 2, 3):
            tgt = (my_i + d) % N_DEV
            pltpu.make_async_remote_copy(
                src_ref=x_ref.at[pl.ds(tgt * BM, BM), :],
                dst_ref=xrecv.at[my_i],
                send_sem=send_sems.at[tgt],
                recv_sem=recv_sems.at[my_i],
                device_id=(tgt,),
                device_id_type=_DeviceIdType.MESH,
            ).wait_send()


def kernel(x, w_mat):
    xb = x.astype(jnp.bfloat16)
    my_i = lax.axis_index("i")
    perm = (my_i + jnp.arange(N_DEV, dtype=jnp.int32)) % N_DEV
    grid_spec = pltpu.PrefetchScalarGridSpec(
        num_scalar_prefetch=1,
        grid=(N_DEV, NN),
        in_specs=[
            pl.BlockSpec(memory_space=pltpu.VMEM),
            pl.BlockSpec((BK, BN), lambda s, n, perm: (perm[s], n)),
        ],
        out_specs=pl.BlockSpec(memory_space=pltpu.VMEM),
        scratch_shapes=[
            pltpu.VMEM((N_DEV, BM, BK), jnp.bfloat16),
            pltpu.SemaphoreType.DMA((N_DEV,)),
            pltpu.SemaphoreType.DMA((N_DEV,)),
            pltpu.SemaphoreType.DMA,
        ],
    )
    return pl.pallas_call(
        _body,
        grid_spec=grid_spec,
        out_shape=jax.ShapeDtypeStruct((BM, N), jnp.float32),
        compiler_params=_CompilerParams(
            collective_id=0,
            dimension_semantics=("arbitrary", "arbitrary"),
            vmem_limit_bytes=62 * 1024 * 1024,
        ),
    )(perm, xb, w_mat)


# device time: 144231 ns/iter; 1.3083x vs baseline; 1.0660x over previous
import jax
import jax.numpy as jnp
from jax import lax
from jax.experimental import pallas as pl
from jax.experimental.pallas import tpu as pltpu

N_DEV = 4
BM = 1024
BK = 1024
N = 8192
BN = 512
NN = N // BN
NPH = 3

_DeviceIdType = getattr(pl, "DeviceIdType", None) or pltpu.DeviceIdType
_sem_signal = getattr(pl, "semaphore_signal", None) or pltpu.semaphore_signal
_sem_wait = getattr(pl, "semaphore_wait", None) or pltpu.semaphore_wait
_CompilerParams = getattr(pltpu, "CompilerParams", None) or pltpu.TPUCompilerParams


def _body(perm_ref, x_ref, wa_ref, wb_ref, out_ref,
          xrecv, send_sems, recv_sems, copy_sem):
    s = pl.program_id(0)
    n = pl.program_id(1)
    my_i = lax.axis_index("i")

    @pl.when(jnp.logical_and(s == 0, n == 0))
    def _init():
        barrier = pltpu.get_barrier_semaphore()
        for d in (1, 2, 3):
            _sem_signal(
                barrier, inc=1,
                device_id=((my_i + d) % N_DEV,),
                device_id_type=_DeviceIdType.MESH,
            )
        _sem_wait(barrier, 3)

        cp = pltpu.make_async_copy(
            x_ref.at[pl.ds(my_i * BM, BM), :], xrecv.at[my_i], copy_sem
        )
        cp.start()

        for d in (3, 1):
            tgt = (my_i + d) % N_DEV
            pltpu.make_async_remote_copy(
                src_ref=x_ref.at[pl.ds(tgt * BM, BM), :],
                dst_ref=xrecv.at[my_i],
                send_sem=send_sems.at[tgt],
                recv_sem=recv_sems.at[my_i],
                device_id=(tgt,),
                device_id_type=_DeviceIdType.MESH,
            ).start()
        cp.wait()

    @pl.when(jnp.logical_and(s == 1, n == 0))
    def _phase1():
        tgt = (my_i + 2) % N_DEV
        pltpu.make_async_remote_copy(
            src_ref=x_ref.at[pl.ds(tgt * BM, BM), :],
            dst_ref=xrecv.at[my_i],
            send_sem=send_sems.at[tgt],
            recv_sem=recv_sems.at[my_i],
            device_id=(tgt,),
            device_id_type=_DeviceIdType.MESH,
        ).start()
        _wait_recv(x_ref, xrecv, send_sems, recv_sems, my_i, perm_ref[1])

    @pl.when(jnp.logical_and(s == 2, n == 0))
    def _phase2():
        _wait_recv(x_ref, xrecv, send_sems, recv_sems, my_i, perm_ref[2])
        _wait_recv(x_ref, xrecv, send_sems, recv_sems, my_i, perm_ref[3])

    cols = pl.ds(n * BN, BN)

    @pl.when(s == 0)
    def _store():
        xb = xrecv[perm_ref[0]]
        wb = wa_ref[...].astype(jnp.bfloat16)
        out_ref[:, cols] = jnp.dot(xb, wb, preferred_element_type=jnp.float32)

    @pl.when(s == 1)
    def _accum1():
        xb = xrecv[perm_ref[1]]
        wb = wa_ref[...].astype(jnp.bfloat16)
        out_ref[:, cols] = out_ref[:, cols] + jnp.dot(
            xb, wb, preferred_element_type=jnp.float32
        )

    @pl.when(s == 2)
    def _accum2():
        x2 = xrecv[perm_ref[2]]
        x3 = xrecv[perm_ref[3]]
        w2 = wa_ref[...].astype(jnp.bfloat16)
        w3 = wb_ref[...].astype(jnp.bfloat16)
        out_ref[:, cols] = out_ref[:, cols] + (
            jnp.dot(x2, w2, preferred_element_type=jnp.float32)
            + jnp.dot(x3, w3, preferred_element_type=jnp.float32)
        )

    @pl.when(jnp.logical_and(s == NPH - 1, n == NN - 1))
    def _fin():
        for d in (1, 2, 3):
            tgt = (my_i + d) % N_DEV
            pltpu.make_async_remote_copy(
                src_ref=x_ref.at[pl.ds(tgt * BM, BM), :],
                dst_ref=xrecv.at[my_i],
                send_sem=send_sems.at[tgt],
                recv_sem=recv_sems.at[my_i],
                device_id=(tgt,),
                device_id_type=_DeviceIdType.MESH,
            ).wait_send()


def _wait_recv(x_ref, xrecv, send_sems, recv_sems, my_i, j):
    pltpu.make_async_remote_copy(
        src_ref=x_ref.at[pl.ds(0, BM), :],
        dst_ref=xrecv.at[j],
        send_sem=send_sems.at[0],
        recv_sem=recv_sems.at[j],
        device_id=(my_i,),
        device_id_type=_DeviceIdType.MESH,
    ).wait_recv()


def kernel(x, w_mat):
    xb = x.astype(jnp.bfloat16)
    my_i = lax.axis_index("i")
    perm = (my_i + jnp.arange(N_DEV, dtype=jnp.int32)) % N_DEV
    grid_spec = pltpu.PrefetchScalarGridSpec(
        num_scalar_prefetch=1,
        grid=(NPH, NN),
        in_specs=[
            pl.BlockSpec(memory_space=pltpu.MemorySpace.HBM),
            pl.BlockSpec((BK, BN), lambda s, n, perm: (perm[s], n)),
            pl.BlockSpec(
                (BK, BN),
                lambda s, n, perm: (perm[3], jnp.where(s == 2, n, 0)),
            ),
        ],
        out_specs=pl.BlockSpec(memory_space=pltpu.VMEM),
        scratch_shapes=[
            pltpu.VMEM((N_DEV, BM, BK), jnp.bfloat16),
            pltpu.SemaphoreType.DMA((N_DEV,)),
            pltpu.SemaphoreType.DMA((N_DEV,)),
            pltpu.SemaphoreType.DMA,
        ],
    )
    return pl.pallas_call(
        _body,
        grid_spec=grid_spec,
        out_shape=jax.ShapeDtypeStruct((BM, N), jnp.float32),
        compiler_params=_CompilerParams(
            collective_id=0,
            dimension_semantics=("arbitrary", "arbitrary"),
            vmem_limit_bytes=62 * 1024 * 1024,
        ),
    )(perm, xb, w_mat, w_mat)


# device time: 129768 ns/iter; 1.4541x vs baseline; 1.1115x over previous
import jax
import jax.numpy as jnp
from jax import lax
from jax.experimental import pallas as pl
from jax.experimental.pallas import tpu as pltpu

N_DEV = 4
BM = 1024
BK = 1024
N = 8192
BN = 512
NN = N // BN
NPH = 3

_DeviceIdType = getattr(pl, "DeviceIdType", None) or pltpu.DeviceIdType
_sem_signal = getattr(pl, "semaphore_signal", None) or pltpu.semaphore_signal
_sem_wait = getattr(pl, "semaphore_wait", None) or pltpu.semaphore_wait
_CompilerParams = getattr(pltpu, "CompilerParams", None) or pltpu.TPUCompilerParams


def _body(perm_ref, x_ref, wa_ref, wb_ref, out_ref,
          xrecv, send_sems, recv_sems, copy_sem):
    s = pl.program_id(0)
    n = pl.program_id(1)
    my_i = lax.axis_index("i")

    @pl.when(jnp.logical_and(s == 0, n == 0))
    def _init():
        barrier = pltpu.get_barrier_semaphore()
        for d in (1, 2, 3):
            _sem_signal(
                barrier, inc=1,
                device_id=((my_i + d) % N_DEV,),
                device_id_type=_DeviceIdType.MESH,
            )
        _sem_wait(barrier, 3)

        cp = pltpu.make_async_copy(
            x_ref.at[pl.ds(my_i * BM, BM), :], xrecv.at[my_i], copy_sem
        )
        cp.start()

        for d in (3, 1):
            tgt = (my_i + d) % N_DEV
            pltpu.make_async_remote_copy(
                src_ref=x_ref.at[pl.ds(tgt * BM, BM), :],
                dst_ref=xrecv.at[my_i],
                send_sem=send_sems.at[tgt],
                recv_sem=recv_sems.at[my_i],
                device_id=(tgt,),
                device_id_type=_DeviceIdType.MESH,
            ).start()
        cp.wait()

    @pl.when(jnp.logical_and(s == 1, n == 0))
    def _phase1():
        tgt = (my_i + 2) % N_DEV
        pltpu.make_async_remote_copy(
            src_ref=x_ref.at[pl.ds(tgt * BM, BM), :],
            dst_ref=xrecv.at[my_i],
            send_sem=send_sems.at[tgt],
            recv_sem=recv_sems.at[my_i],
            device_id=(tgt,),
            device_id_type=_DeviceIdType.MESH,
        ).start()
        _wait_recv(x_ref, xrecv, send_sems, recv_sems, my_i, perm_ref[1])

    @pl.when(jnp.logical_and(s == 2, n == 0))
    def _phase2():
        _wait_recv(x_ref, xrecv, send_sems, recv_sems, my_i, perm_ref[2])
        _wait_recv(x_ref, xrecv, send_sems, recv_sems, my_i, perm_ref[3])

    cols = pl.ds(n * BN, BN)

    @pl.when(s == 0)
    def _store():
        xb = xrecv[perm_ref[0]]
        wb = wa_ref[...].astype(jnp.bfloat16)
        out_ref[:, cols] = jnp.dot(
            xb, wb, preferred_element_type=jnp.float32
        ).astype(jnp.bfloat16)

    @pl.when(s == 1)
    def _accum1():
        xb = xrecv[perm_ref[1]]
        wb = wa_ref[...].astype(jnp.bfloat16)
        out_ref[:, cols] = (
            out_ref[:, cols].astype(jnp.float32)
            + jnp.dot(xb, wb, preferred_element_type=jnp.float32)
        ).astype(jnp.bfloat16)

    @pl.when(s == 2)
    def _accum2():
        x2 = xrecv[perm_ref[2]]
        x3 = xrecv[perm_ref[3]]
        w2 = wa_ref[...].astype(jnp.bfloat16)
        w3 = wb_ref[...].astype(jnp.bfloat16)
        out_ref[:, cols] = (
            out_ref[:, cols].astype(jnp.float32)
            + jnp.dot(x2, w2, preferred_element_type=jnp.float32)
            + jnp.dot(x3, w3, preferred_element_type=jnp.float32)
        ).astype(jnp.bfloat16)

    @pl.when(jnp.logical_and(s == NPH - 1, n == NN - 1))
    def _fin():
        for d in (1, 2, 3):
            tgt = (my_i + d) % N_DEV
            pltpu.make_async_remote_copy(
                src_ref=x_ref.at[pl.ds(tgt * BM, BM), :],
                dst_ref=xrecv.at[my_i],
                send_sem=send_sems.at[tgt],
                recv_sem=recv_sems.at[my_i],
                device_id=(tgt,),
                device_id_type=_DeviceIdType.MESH,
            ).wait_send()


def _wait_recv(x_ref, xrecv, send_sems, recv_sems, my_i, j):
    pltpu.make_async_remote_copy(
        src_ref=x_ref.at[pl.ds(0, BM), :],
        dst_ref=xrecv.at[j],
        send_sem=send_sems.at[0],
        recv_sem=recv_sems.at[j],
        device_id=(my_i,),
        device_id_type=_DeviceIdType.MESH,
    ).wait_recv()


def kernel(x, w_mat):
    xb = x.astype(jnp.bfloat16)
    my_i = lax.axis_index("i")
    perm = (my_i + jnp.arange(N_DEV, dtype=jnp.int32)) % N_DEV
    grid_spec = pltpu.PrefetchScalarGridSpec(
        num_scalar_prefetch=1,
        grid=(NPH, NN),
        in_specs=[
            pl.BlockSpec(memory_space=pltpu.MemorySpace.HBM),
            pl.BlockSpec((BK, BN), lambda s, n, perm: (perm[s], n)),
            pl.BlockSpec(
                (BK, BN),
                lambda s, n, perm: (perm[3], jnp.where(s == 2, n, 0)),
            ),
        ],
        out_specs=pl.BlockSpec(memory_space=pltpu.VMEM),
        scratch_shapes=[
            pltpu.VMEM((N_DEV, BM, BK), jnp.bfloat16),
            pltpu.SemaphoreType.DMA((N_DEV,)),
            pltpu.SemaphoreType.DMA((N_DEV,)),
            pltpu.SemaphoreType.DMA,
        ],
    )
    return pl.pallas_call(
        _body,
        grid_spec=grid_spec,
        out_shape=jax.ShapeDtypeStruct((BM, N), jnp.bfloat16),
        compiler_params=_CompilerParams(
            collective_id=0,
            dimension_semantics=("arbitrary", "arbitrary"),
            vmem_limit_bytes=62 * 1024 * 1024,
        ),
    )(perm, xb, w_mat, w_mat)


# device time: 118943 ns/iter; 1.5864x vs baseline; 1.0910x over previous
import jax
import jax.numpy as jnp
from jax import lax
from jax.experimental import pallas as pl
from jax.experimental.pallas import tpu as pltpu

N_DEV = 4
BM = 1024
BK = 1024
N = 8192
BN = 1024
NN = N // BN
NPH = 3

_DeviceIdType = getattr(pl, "DeviceIdType", None) or pltpu.DeviceIdType
_sem_signal = getattr(pl, "semaphore_signal", None) or pltpu.semaphore_signal
_sem_wait = getattr(pl, "semaphore_wait", None) or pltpu.semaphore_wait
_CompilerParams = getattr(pltpu, "CompilerParams", None) or pltpu.TPUCompilerParams


def _body(perm_ref, x_ref, wa_ref, wb_ref, out_ref,
          xrecv, acc, send_sems, recv_sems, copy_sem, out_sems):
    s = pl.program_id(0)
    n = pl.program_id(1)
    my_i = lax.axis_index("i")

    @pl.when(jnp.logical_and(s == 0, n == 0))
    def _init():
        barrier = pltpu.get_barrier_semaphore()
        for d in (1, 2, 3):
            _sem_signal(
                barrier, inc=1,
                device_id=((my_i + d) % N_DEV,),
                device_id_type=_DeviceIdType.MESH,
            )
        _sem_wait(barrier, 3)

        cp = pltpu.make_async_copy(
            x_ref.at[pl.ds(my_i * BM, BM), :], xrecv.at[my_i], copy_sem
        )
        cp.start()

        for d in (3, 1):
            tgt = (my_i + d) % N_DEV
            pltpu.make_async_remote_copy(
                src_ref=x_ref.at[pl.ds(tgt * BM, BM), :],
                dst_ref=xrecv.at[my_i],
                send_sem=send_sems.at[tgt],
                recv_sem=recv_sems.at[my_i],
                device_id=(tgt,),
                device_id_type=_DeviceIdType.MESH,
            ).start()
        cp.wait()

    @pl.when(jnp.logical_and(s == 1, n == 0))
    def _phase1():
        tgt = (my_i + 2) % N_DEV
        pltpu.make_async_remote_copy(
            src_ref=x_ref.at[pl.ds(tgt * BM, BM), :],
            dst_ref=xrecv.at[my_i],
            send_sem=send_sems.at[tgt],
            recv_sem=recv_sems.at[my_i],
            device_id=(tgt,),
            device_id_type=_DeviceIdType.MESH,
        ).start()
        _wait_recv(x_ref, xrecv, send_sems, recv_sems, my_i, perm_ref[1])

    @pl.when(jnp.logical_and(s == 2, n == 0))
    def _phase2():
        _wait_recv(x_ref, xrecv, send_sems, recv_sems, my_i, perm_ref[2])
        _wait_recv(x_ref, xrecv, send_sems, recv_sems, my_i, perm_ref[3])

    cols = pl.ds(n * BN, BN)

    @pl.when(s == 0)
    def _store():
        xb = xrecv[perm_ref[0]]
        wb = wa_ref[...].astype(jnp.bfloat16)
        acc[:, cols] = jnp.dot(
            xb, wb, preferred_element_type=jnp.float32
        ).astype(jnp.bfloat16)

    @pl.when(s == 1)
    def _accum1():
        xb = xrecv[perm_ref[1]]
        wb = wa_ref[...].astype(jnp.bfloat16)
        acc[:, cols] = (
            acc[:, cols].astype(jnp.float32)
            + jnp.dot(xb, wb, preferred_element_type=jnp.float32)
        ).astype(jnp.bfloat16)

    @pl.when(s == 2)
    def _accum2():
        x2 = xrecv[perm_ref[2]]
        x3 = xrecv[perm_ref[3]]
        w2 = wa_ref[...].astype(jnp.bfloat16)
        w3 = wb_ref[...].astype(jnp.bfloat16)
        acc[:, cols] = (
            acc[:, cols].astype(jnp.float32)
            + jnp.dot(x2, w2, preferred_element_type=jnp.float32)
            + jnp.dot(x3, w3, preferred_element_type=jnp.float32)
        ).astype(jnp.bfloat16)
        pltpu.make_async_copy(
            acc.at[:, cols], out_ref.at[:, cols], out_sems.at[n]
        ).start()

    @pl.when(jnp.logical_and(s == NPH - 1, n == NN - 1))
    def _fin():
        for k in range(NN):
            pltpu.make_async_copy(
                acc.at[:, pl.ds(k * BN, BN)],
                out_ref.at[:, pl.ds(k * BN, BN)],
                out_sems.at[k],
            ).wait()
        for d in (1, 2, 3):
            tgt = (my_i + d) % N_DEV
            pltpu.make_async_remote_copy(
                src_ref=x_ref.at[pl.ds(tgt * BM, BM), :],
                dst_ref=xrecv.at[my_i],
                send_sem=send_sems.at[tgt],
                recv_sem=recv_sems.at[my_i],
                device_id=(tgt,),
                device_id_type=_DeviceIdType.MESH,
            ).wait_send()


def _wait_recv(x_ref, xrecv, send_sems, recv_sems, my_i, j):
    pltpu.make_async_remote_copy(
        src_ref=x_ref.at[pl.ds(0, BM), :],
        dst_ref=xrecv.at[j],
        send_sem=send_sems.at[0],
        recv_sem=recv_sems.at[j],
        device_id=(my_i,),
        device_id_type=_DeviceIdType.MESH,
    ).wait_recv()


def kernel(x, w_mat):
    xb = x.astype(jnp.bfloat16)
    my_i = lax.axis_index("i")
    perm = (my_i + jnp.arange(N_DEV, dtype=jnp.int32)) % N_DEV
    grid_spec = pltpu.PrefetchScalarGridSpec(
        num_scalar_prefetch=1,
        grid=(NPH, NN),
        in_specs=[
            pl.BlockSpec(memory_space=pltpu.MemorySpace.HBM),
            pl.BlockSpec((BK, BN), lambda s, n, perm: (perm[s], n)),
            pl.BlockSpec(
                (BK, BN),
                lambda s, n, perm: (perm[3], jnp.where(s == 2, n, 0)),
            ),
        ],
        out_specs=pl.BlockSpec(memory_space=pltpu.MemorySpace.HBM),
        scratch_shapes=[
            pltpu.VMEM((N_DEV, BM, BK), jnp.bfloat16),
            pltpu.VMEM((BM, N), jnp.bfloat16),
            pltpu.SemaphoreType.DMA((N_DEV,)),
            pltpu.SemaphoreType.DMA((N_DEV,)),
            pltpu.SemaphoreType.DMA,
            pltpu.SemaphoreType.DMA((NN,)),
        ],
    )
    return pl.pallas_call(
        _body,
        grid_spec=grid_spec,
        out_shape=jax.ShapeDtypeStruct((BM, N), jnp.bfloat16),
        compiler_params=_CompilerParams(
            collective_id=0,
            dimension_semantics=("arbitrary", "arbitrary"),
            vmem_limit_bytes=62 * 1024 * 1024,
        ),
    )(perm, xb, w_mat, w_mat)
